# baseline (speedup 1.0000x reference)
import functools

import jax
import jax.numpy as jnp
from jax import lax
from jax.experimental import pallas as pl
from jax.experimental.pallas import tpu as pltpu
from jax.experimental.pallas import tpu_sc as plsc

H = 256


def _ln(x, g, b, eps=1e-5):
    m = jnp.mean(x, axis=-1, keepdims=True)
    v = jnp.mean((x - m) ** 2, axis=-1, keepdims=True)
    return (x - m) * jax.lax.rsqrt(v + eps) * g + b


def _edge_mlp_body(ef_ref, rf_ref, nf_ref, We_ref, be_ref, out_ref):
    acc = jnp.dot(ef_ref[...], We_ref[0:H, :], preferred_element_type=jnp.float32)
    acc += jnp.dot(rf_ref[...], We_ref[H:2 * H, :], preferred_element_type=jnp.float32)
    acc += jnp.dot(nf_ref[...], We_ref[2 * H:3 * H, :], preferred_element_type=jnp.float32)
    out_ref[...] = jnp.maximum(acc + be_ref[...], 0.0)


def _edge_mlp(ef, rf, nf, We, be, block_rows):
    E = ef.shape[0]
    assert E % block_rows == 0
    grid = (E // block_rows,)
    row_spec = pl.BlockSpec((block_rows, H), lambda i: (i, 0))
    full_w = pl.BlockSpec((3 * H, H), lambda i: (0, 0))
    vec = pl.BlockSpec((H,), lambda i: (0,))
    return pl.pallas_call(
        _edge_mlp_body,
        grid=grid,
        in_specs=[row_spec, row_spec, row_spec, full_w, vec],
        out_specs=row_spec,
        out_shape=jax.ShapeDtypeStruct((E, H), jnp.float32),
    )(ef, rf, nf, We, be)


def _node_mlp_block(nf, agg, W1_ref, b1_ref, g1_ref, bt1_ref, W2_ref, b2_ref,
                    g2_ref, bt2_ref):
    m = jnp.dot(nf, W1_ref[0:H, :], preferred_element_type=jnp.float32)
    m += jnp.dot(agg, W1_ref[H:2 * H, :], preferred_element_type=jnp.float32)
    h = jnp.maximum(_ln(m + b1_ref[...], g1_ref[...], bt1_ref[...]), 0.0)
    h2 = jnp.dot(h, W2_ref[...], preferred_element_type=jnp.float32) + b2_ref[...]
    return jnp.maximum(_ln(h2, g2_ref[...], bt2_ref[...]), 0.0)


def _node_edge_body(nf1_ref, agg_ref, ef_ref, rf_ref,
                    W1_ref, b1_ref, g1_ref, bt1_ref, W2_ref, b2_ref, g2_ref,
                    bt2_ref, We_ref, be_ref, out_ref):
    n1 = _node_mlp_block(nf1_ref[...], agg_ref[...], W1_ref, b1_ref, g1_ref,
                         bt1_ref, W2_ref, b2_ref, g2_ref, bt2_ref)
    acc = jnp.dot(ef_ref[...], We_ref[0:H, :], preferred_element_type=jnp.float32)
    acc += jnp.dot(rf_ref[...], We_ref[H:2 * H, :], preferred_element_type=jnp.float32)
    acc += jnp.dot(n1, We_ref[2 * H:3 * H, :], preferred_element_type=jnp.float32)
    out_ref[...] = jnp.maximum(acc + be_ref[...], 0.0)


def _node_edge_mlp(nf1, agg, ef, rf, W1, b1, g1, bt1, W2, b2, g2, bt2, We, be,
                   block_rows):
    S = nf1.shape[0]
    assert S % block_rows == 0
    grid = (S // block_rows,)
    row_spec = pl.BlockSpec((block_rows, H), lambda i: (i, 0))
    w2h = pl.BlockSpec((2 * H, H), lambda i: (0, 0))
    w1h = pl.BlockSpec((H, H), lambda i: (0, 0))
    w3h = pl.BlockSpec((3 * H, H), lambda i: (0, 0))
    vec = pl.BlockSpec((H,), lambda i: (0,))
    return pl.pallas_call(
        _node_edge_body,
        grid=grid,
        in_specs=[row_spec, row_spec, row_spec, row_spec,
                  w2h, vec, vec, vec, w1h, vec, vec, vec, w3h, vec],
        out_specs=row_spec,
        out_shape=jax.ShapeDtypeStruct((S, H), jnp.float32),
    )(nf1, agg, ef, rf, W1, b1, g1, bt1, W2, b2, g2, bt2, We, be)


def _node_body(nf_ref, agg_ref, W1_ref, b1_ref, g1_ref, bt1_ref, W2_ref,
               b2_ref, g2_ref, bt2_ref, out_ref):
    out_ref[...] = _node_mlp_block(nf_ref[...], agg_ref[...], W1_ref, b1_ref,
                                   g1_ref, bt1_ref, W2_ref, b2_ref, g2_ref,
                                   bt2_ref)


def _node_mlp(nf, agg, W1, b1, g1, bt1, W2, b2, g2, bt2, block_rows):
    S = nf.shape[0]
    assert S % block_rows == 0
    grid = (S // block_rows,)
    row_spec = pl.BlockSpec((block_rows, H), lambda i: (i, 0))
    w2h = pl.BlockSpec((2 * H, H), lambda i: (0, 0))
    w1h = pl.BlockSpec((H, H), lambda i: (0, 0))
    vec = pl.BlockSpec((H,), lambda i: (0,))
    return pl.pallas_call(
        _node_body,
        grid=grid,
        in_specs=[row_spec, row_spec, w2h, vec, vec, vec, w1h, vec, vec, vec],
        out_specs=row_spec,
        out_shape=jax.ShapeDtypeStruct((S, H), jnp.float32),
    )(nf, agg, W1, b1, g1, bt1, W2, b2, g2, bt2)


def _make_seg_sum_sc(E, E_pad, S, G=64, SUB=224):
    NT = 32
    SLAB = E_pad // 16
    n_vregs = SLAB // 16
    S_pad = -(-S // 256) * 256
    OWN = S_pad // NT
    while OWN % SUB:
        SUB -= 8
    n_sub = OWN // SUB
    SH = 18
    MASK18 = (1 << SH) - 1
    DUMPLOC = (1 << 13) - 1
    DUMPPACK = DUMPLOC << SH
    C = 256
    LBUF = (SLAB // C) * C
    LREG = 16 * (SLAB + C) + LBUF
    BUFCAP = max(SLAB + C, LBUF + C)

    assert E_pad % 256 == 0 and E_pad <= (1 << SH)
    assert OWN % 8 == 0 and SUB % 8 == 0 and OWN + 1 < DUMPLOC

    mesh = plsc.VectorSubcoreMesh(core_axis_name="c", subcore_axis_name="s")

    @functools.partial(
        pl.kernel,
        out_type=(jax.ShapeDtypeStruct((S_pad * H,), jnp.float32),
                  jax.ShapeDtypeStruct((NT * LREG,), jnp.int32)),
        mesh=mesh,
        scratch_types=[
            pltpu.VMEM((SLAB,), jnp.int32),
            pltpu.VMEM((LBUF,), jnp.int32),
            pltpu.VMEM((SLAB + C,), jnp.int32),
            pltpu.VMEM((G,), jnp.int32),
            pltpu.VMEM((G, H), jnp.float32),
            pltpu.VMEM((SUB * H,), jnp.float32),
            pltpu.SemaphoreType.DMA,
        ],
        compiler_params=pltpu.CompilerParams(needs_layout_passes=False),
    )
    def seg(x_hbm, dst_hbm, out_hbm, list_hbm, dstbuf, listbuf, bufB,
            idxstage, rowbuf, acc, sem):
        c = lax.axis_index("c")
        s = lax.axis_index("s")
        w = c * 16 + s
        lo = w * OWN
        myreg = w * LREG

        iota = lax.iota(jnp.int32, 16)
        zi = jnp.zeros((16,), jnp.int32)
        zf = jnp.zeros((16,), jnp.float32)
        dumpvec = jnp.full((16,), DUMPPACK, jnp.int32)

        def _chunk(ch, gcur):
            pltpu.sync_copy(dst_hbm.at[pl.ds(ch * SLAB, SLAB)], dstbuf)

            def _cbody(v, cur):
                d = dstbuf[pl.ds(v * 16, 16)]
                m = (d >= lo) & (d < lo + OWN)
                mi = jnp.where(m, jnp.int32(1), jnp.int32(0))
                csum = plsc.cumsum(mi)
                pos = cur + csum - 1
                eidx = iota + (ch * SLAB + v * 16)
                packed = eidx | ((d - lo) << SH)
                plsc.store_scatter(bufB, [pos], packed, mask=m)
                return cur + jnp.max(csum)
            cur = lax.fori_loop(0, n_vregs, _cbody, jnp.int32(0))

            for k in range(C // 16):
                bufB[pl.ds(cur + k * 16, 16)] = dumpvec
            n_c = (cur + C - 1) // C

            def _flush(i, _):
                pltpu.sync_copy(
                    bufB.at[pl.ds(i * C, C)],
                    list_hbm.at[pl.ds(
                        pl.multiple_of(myreg + gcur + i * C, C), C)])
                return 0
            lax.fori_loop(0, n_c, _flush, 0)
            return gcur + n_c * C
        gcur = lax.fori_loop(0, 16, _chunk, jnp.int32(0))

        gv = zi + gcur

        sub_size = SUB
        def _sub(p, _):
            sub_base = p * SUB

            def _zacc(i, _):
                for k in range(H // 16):
                    acc[pl.ds(i * H + k * 16, 16)] = zf
                return 0
            lax.fori_loop(0, sub_size, _zacc, 0)

            n_l = (gcur + LBUF - 1) // LBUF

            def _lchunk(i, _):
                pltpu.sync_copy(
                    list_hbm.at[pl.ds(
                        pl.multiple_of(myreg + i * LBUF, C), LBUF)],
                    listbuf)

                def _sbody(v, cur):
                    pk = listbuf[pl.ds(v * 16, 16)]
                    lvt = pk >> SH
                    ei = iota + (i * LBUF + v * 16)
                    m = ((lvt >= sub_base) & (lvt < sub_base + sub_size)
                         & (ei < gv))
                    mi = jnp.where(m, jnp.int32(1), jnp.int32(0))
                    csum = plsc.cumsum(mi)
                    pos = cur + csum - 1
                    plsc.store_scatter(bufB, [pos], pk, mask=m)
                    return cur + jnp.max(csum)
                cur = lax.fori_loop(0, LBUF // 16, _sbody, jnp.int32(0))

                for k in range(G // 16):
                    bufB[pl.ds(cur + k * 16, 16)] = dumpvec
                n_g = (cur + G - 1) // G

                def _gbody(g, _):
                    for k in range(G // 16):
                        pk = bufB[pl.ds(g * G + k * 16, 16)]
                        idxstage[pl.ds(k * 16, 16)] = pk & MASK18
                    pltpu.async_copy(x_hbm.at[idxstage], rowbuf, sem).wait()

                    def _grp(k, _):
                        pk16 = bufB[pl.ds(g * G + k * 16, 16)]
                        lv16 = (pk16 >> SH) - sub_base
                        for e in range(16):
                            lv = lv16[e]

                            @pl.when(lv < sub_size)
                            def _():
                                base = pl.multiple_of(lv * H, H)
                                row = k * 16 + e
                                for cc in range(H // 16):
                                    sl = pl.ds(base + cc * 16, 16)
                                    acc[sl] += rowbuf[row, pl.ds(cc * 16, 16)]
                        return 0
                    lax.fori_loop(0, G // 16, _grp, 0)
                    return 0
                lax.fori_loop(0, n_g, _gbody, 0)
                return 0
            lax.fori_loop(0, n_l, _lchunk, 0)

            pltpu.sync_copy(
                acc.at[pl.ds(0, sub_size * H)],
                out_hbm.at[pl.ds(
                    pl.multiple_of((lo + sub_base) * H, H), sub_size * H)])
            return 0
        lax.fori_loop(0, n_sub, _sub, 0)

    return seg, S_pad


def _segment_sum(x, dst, num_segments):
    E = x.shape[0]
    E_pad = -(-E // 256) * 256
    seg, S_pad = _make_seg_sum_sc(E, E_pad, num_segments)
    dst_pad = jnp.concatenate(
        [dst, jnp.full((E_pad - E,), S_pad, jnp.int32)]) if E_pad > E else dst
    out, _ = seg(x, dst_pad)
    return out.reshape(S_pad, H)[:num_segments]


def kernel(n_feat_0, n_feat_1, n_feat_2, e_feat_1, e_feat_2, r_feat_1,
           r_feat_2, dst_1, dst_2, We_1, be_1, W1_1, b1_1, g1_1, bt1_1, W2_1,
           b2_1, g2_1, bt2_1, We_2, be_2, W1_2, b1_2, g1_2, bt1_2, W2_2, b2_2,
           g2_2, bt2_2):
    N0, N1, N2 = n_feat_0.shape[0], n_feat_1.shape[0], n_feat_2.shape[0]

    e_repr_2 = _edge_mlp(e_feat_2, r_feat_2, n_feat_2, We_2, be_2,
                         block_rows=1000)
    agg_2 = _segment_sum(e_repr_2, dst_2, N1)
    e_repr_1 = _node_edge_mlp(n_feat_1, agg_2, e_feat_1, r_feat_1,
                              W1_2, b1_2, g1_2, bt1_2, W2_2, b2_2, g2_2, bt2_2,
                              We_1, be_1, block_rows=1000)
    agg_1 = _segment_sum(e_repr_1, dst_1, N0)
    n0 = _node_mlp(n_feat_0, agg_1, W1_1, b1_1, g1_1, bt1_1, W2_1, b2_1,
                   g2_1, bt2_1, block_rows=1000)
    return n0


# speedup vs baseline: 1.0365x; 1.0365x over previous
import functools

import jax
import jax.numpy as jnp
from jax import lax
from jax.experimental import pallas as pl
from jax.experimental.pallas import tpu as pltpu
from jax.experimental.pallas import tpu_sc as plsc

H = 256


def _ln(x, g, b, eps=1e-5):
    m = jnp.mean(x, axis=-1, keepdims=True)
    v = jnp.mean((x - m) ** 2, axis=-1, keepdims=True)
    return (x - m) * jax.lax.rsqrt(v + eps) * g + b


def _edge_mlp_body(ef_ref, rf_ref, nf_ref, We_ref, be_ref, out_ref):
    acc = jnp.dot(ef_ref[...], We_ref[0:H, :], preferred_element_type=jnp.float32)
    acc += jnp.dot(rf_ref[...], We_ref[H:2 * H, :], preferred_element_type=jnp.float32)
    acc += jnp.dot(nf_ref[...], We_ref[2 * H:3 * H, :], preferred_element_type=jnp.float32)
    out_ref[...] = jnp.maximum(acc + be_ref[...], 0.0)


def _edge_mlp(ef, rf, nf, We, be, block_rows):
    E = ef.shape[0]
    assert E % block_rows == 0
    grid = (E // block_rows,)
    row_spec = pl.BlockSpec((block_rows, H), lambda i: (i, 0))
    full_w = pl.BlockSpec((3 * H, H), lambda i: (0, 0))
    vec = pl.BlockSpec((H,), lambda i: (0,))
    return pl.pallas_call(
        _edge_mlp_body,
        grid=grid,
        in_specs=[row_spec, row_spec, row_spec, full_w, vec],
        out_specs=row_spec,
        out_shape=jax.ShapeDtypeStruct((E, H), jnp.float32),
    )(ef, rf, nf, We, be)


def _node_mlp_block(nf, agg, W1_ref, b1_ref, g1_ref, bt1_ref, W2_ref, b2_ref,
                    g2_ref, bt2_ref):
    m = jnp.dot(nf, W1_ref[0:H, :], preferred_element_type=jnp.float32)
    m += jnp.dot(agg, W1_ref[H:2 * H, :], preferred_element_type=jnp.float32)
    h = jnp.maximum(_ln(m + b1_ref[...], g1_ref[...], bt1_ref[...]), 0.0)
    h2 = jnp.dot(h, W2_ref[...], preferred_element_type=jnp.float32) + b2_ref[...]
    return jnp.maximum(_ln(h2, g2_ref[...], bt2_ref[...]), 0.0)


def _node_edge_body(nf1_ref, agg_ref, ef_ref, rf_ref,
                    W1_ref, b1_ref, g1_ref, bt1_ref, W2_ref, b2_ref, g2_ref,
                    bt2_ref, We_ref, be_ref, out_ref):
    n1 = _node_mlp_block(nf1_ref[...], agg_ref[...], W1_ref, b1_ref, g1_ref,
                         bt1_ref, W2_ref, b2_ref, g2_ref, bt2_ref)
    acc = jnp.dot(ef_ref[...], We_ref[0:H, :], preferred_element_type=jnp.float32)
    acc += jnp.dot(rf_ref[...], We_ref[H:2 * H, :], preferred_element_type=jnp.float32)
    acc += jnp.dot(n1, We_ref[2 * H:3 * H, :], preferred_element_type=jnp.float32)
    out_ref[...] = jnp.maximum(acc + be_ref[...], 0.0)


def _node_edge_mlp(nf1, agg, ef, rf, W1, b1, g1, bt1, W2, b2, g2, bt2, We, be,
                   block_rows):
    S = nf1.shape[0]
    assert S % block_rows == 0
    grid = (S // block_rows,)
    row_spec = pl.BlockSpec((block_rows, H), lambda i: (i, 0))
    w2h = pl.BlockSpec((2 * H, H), lambda i: (0, 0))
    w1h = pl.BlockSpec((H, H), lambda i: (0, 0))
    w3h = pl.BlockSpec((3 * H, H), lambda i: (0, 0))
    vec = pl.BlockSpec((H,), lambda i: (0,))
    return pl.pallas_call(
        _node_edge_body,
        grid=grid,
        in_specs=[row_spec, row_spec, row_spec, row_spec,
                  w2h, vec, vec, vec, w1h, vec, vec, vec, w3h, vec],
        out_specs=row_spec,
        out_shape=jax.ShapeDtypeStruct((S, H), jnp.float32),
    )(nf1, agg, ef, rf, W1, b1, g1, bt1, W2, b2, g2, bt2, We, be)


def _node_body(nf_ref, agg_ref, W1_ref, b1_ref, g1_ref, bt1_ref, W2_ref,
               b2_ref, g2_ref, bt2_ref, out_ref):
    out_ref[...] = _node_mlp_block(nf_ref[...], agg_ref[...], W1_ref, b1_ref,
                                   g1_ref, bt1_ref, W2_ref, b2_ref, g2_ref,
                                   bt2_ref)


def _node_mlp(nf, agg, W1, b1, g1, bt1, W2, b2, g2, bt2, block_rows):
    S = nf.shape[0]
    assert S % block_rows == 0
    grid = (S // block_rows,)
    row_spec = pl.BlockSpec((block_rows, H), lambda i: (i, 0))
    w2h = pl.BlockSpec((2 * H, H), lambda i: (0, 0))
    w1h = pl.BlockSpec((H, H), lambda i: (0, 0))
    vec = pl.BlockSpec((H,), lambda i: (0,))
    return pl.pallas_call(
        _node_body,
        grid=grid,
        in_specs=[row_spec, row_spec, w2h, vec, vec, vec, w1h, vec, vec, vec],
        out_specs=row_spec,
        out_shape=jax.ShapeDtypeStruct((S, H), jnp.float32),
    )(nf, agg, W1, b1, g1, bt1, W2, b2, g2, bt2)


def _make_seg_sum_sc(E, E_pad, S, G=64, SUB=224):
    NT = 32
    SLAB = E_pad // 16
    n_vregs = SLAB // 16
    S_pad = -(-S // 256) * 256
    OWN = S_pad // NT
    while OWN % SUB:
        SUB -= 8
    n_sub = OWN // SUB
    SH = 18
    MASK18 = (1 << SH) - 1
    DUMPLOC = (1 << 13) - 1
    DUMPPACK = DUMPLOC << SH
    C = 256
    LBUF = (SLAB // C) * C
    LREG = 16 * (SLAB + C) + LBUF
    BUFCAP = max(SLAB + C, LBUF + C)

    assert E_pad % 256 == 0 and E_pad <= (1 << SH)
    assert OWN % 8 == 0 and SUB % 8 == 0 and OWN + 1 < DUMPLOC

    mesh = plsc.VectorSubcoreMesh(core_axis_name="c", subcore_axis_name="s")

    @functools.partial(
        pl.kernel,
        out_type=(jax.ShapeDtypeStruct((S_pad * H,), jnp.float32),
                  jax.ShapeDtypeStruct((NT * LREG,), jnp.int32)),
        mesh=mesh,
        scratch_types=[
            pltpu.VMEM((SLAB,), jnp.int32),
            pltpu.VMEM((LBUF,), jnp.int32),
            pltpu.VMEM((SLAB + C,), jnp.int32),
            pltpu.VMEM((G,), jnp.int32),
            pltpu.VMEM((G, H), jnp.float32),
            pltpu.VMEM(((SUB + 8) * H,), jnp.float32),
            pltpu.SemaphoreType.DMA,
        ],
        compiler_params=pltpu.CompilerParams(needs_layout_passes=False),
    )
    def seg(x_hbm, dst_hbm, out_hbm, list_hbm, dstbuf, listbuf, bufB,
            idxstage, rowbuf, acc, sem):
        c = lax.axis_index("c")
        s = lax.axis_index("s")
        w = c * 16 + s
        lo = w * OWN
        myreg = w * LREG

        iota = lax.iota(jnp.int32, 16)
        fifteen = jnp.full((16,), 15, jnp.int32)
        zi = jnp.zeros((16,), jnp.int32)
        zf = jnp.zeros((16,), jnp.float32)
        dumpvec = jnp.full((16,), DUMPPACK, jnp.int32)

        def _chunk(ch, gcur):
            pltpu.sync_copy(dst_hbm.at[pl.ds(ch * SLAB, SLAB)], dstbuf)

            def _cbody(v, cur):
                d = dstbuf[pl.ds(v * 16, 16)]
                m = (d >= lo) & (d < lo + OWN)
                mi = jnp.where(m, jnp.int32(1), jnp.int32(0))
                csum = plsc.cumsum(mi)
                pos = cur + csum - 1
                eidx = iota + (ch * SLAB + v * 16)
                packed = eidx | ((d - lo) << SH)
                plsc.store_scatter(bufB, [pos], packed, mask=m)
                return cur + jnp.max(csum)
            cur = lax.fori_loop(0, n_vregs, _cbody, jnp.int32(0))

            for k in range(C // 16):
                bufB[pl.ds(cur + k * 16, 16)] = dumpvec
            n_c = (cur + C - 1) // C

            def _flush(i, _):
                pltpu.sync_copy(
                    bufB.at[pl.ds(i * C, C)],
                    list_hbm.at[pl.ds(
                        pl.multiple_of(myreg + gcur + i * C, C), C)])
                return 0
            lax.fori_loop(0, n_c, _flush, 0)
            return gcur + n_c * C
        gcur = lax.fori_loop(0, 16, _chunk, jnp.int32(0))

        gv = zi + gcur

        sub_size = SUB
        def _sub(p, _):
            sub_base = p * SUB

            def _zacc(i, _):
                for k in range(H // 16):
                    acc[pl.ds(i * H + k * 16, 16)] = zf
                return 0
            lax.fori_loop(0, sub_size, _zacc, 0)

            n_l = (gcur + LBUF - 1) // LBUF

            def _lchunk(i, _):
                pltpu.sync_copy(
                    list_hbm.at[pl.ds(
                        pl.multiple_of(myreg + i * LBUF, C), LBUF)],
                    listbuf)

                def _sbody(v, cur):
                    pk = listbuf[pl.ds(v * 16, 16)]
                    lvt = pk >> SH
                    ei = iota + (i * LBUF + v * 16)
                    m = ((lvt >= sub_base) & (lvt < sub_base + sub_size)
                         & (ei < gv))
                    mi = jnp.where(m, jnp.int32(1), jnp.int32(0))
                    csum = plsc.cumsum(mi)
                    pos = cur + csum - 1
                    plsc.store_scatter(bufB, [pos], pk, mask=m)
                    return cur + jnp.max(csum)
                cur = lax.fori_loop(0, LBUF // 16, _sbody, jnp.int32(0))

                for k in range(G // 16):
                    bufB[pl.ds(cur + k * 16, 16)] = dumpvec
                n_g = (cur + G - 1) // G

                def _gbody(g, _):
                    for k in range(G // 16):
                        pk = bufB[pl.ds(g * G + k * 16, 16)]
                        idxstage[pl.ds(k * 16, 16)] = pk & MASK18
                    pltpu.async_copy(x_hbm.at[idxstage], rowbuf, sem).wait()

                    def _grp(k, _):
                        pk16 = bufB[pl.ds(g * G + k * 16, 16)]
                        lv16 = (pk16 >> SH) - sub_base
                        lv16 = jnp.where(
                            (lv16 >= 0) & (lv16 < sub_size), lv16, SUB)
                        for e in range(16):
                            lv = lv16[e]
                            base = pl.multiple_of(lv * H, H)
                            row = k * 16 + e
                            for cc in range(H // 16):
                                sl = pl.ds(base + cc * 16, 16)
                                acc[sl] += rowbuf[row, pl.ds(cc * 16, 16)]
                        return 0
                    lax.fori_loop(0, G // 16, _grp, 0)
                    return 0
                lax.fori_loop(0, n_g, _gbody, 0)
                return 0
            lax.fori_loop(0, n_l, _lchunk, 0)

            pltpu.sync_copy(
                acc.at[pl.ds(0, sub_size * H)],
                out_hbm.at[pl.ds(
                    pl.multiple_of((lo + sub_base) * H, H), sub_size * H)])
            return 0
        lax.fori_loop(0, n_sub, _sub, 0)

    return seg, S_pad


def _segment_sum(x, dst, num_segments):
    E = x.shape[0]
    E_pad = -(-E // 256) * 256
    seg, S_pad = _make_seg_sum_sc(E, E_pad, num_segments)
    dst_pad = jnp.concatenate(
        [dst, jnp.full((E_pad - E,), S_pad, jnp.int32)]) if E_pad > E else dst
    out, _ = seg(x, dst_pad)
    return out.reshape(S_pad, H)[:num_segments]


def kernel(n_feat_0, n_feat_1, n_feat_2, e_feat_1, e_feat_2, r_feat_1,
           r_feat_2, dst_1, dst_2, We_1, be_1, W1_1, b1_1, g1_1, bt1_1, W2_1,
           b2_1, g2_1, bt2_1, We_2, be_2, W1_2, b1_2, g1_2, bt1_2, W2_2, b2_2,
           g2_2, bt2_2):
    N0, N1, N2 = n_feat_0.shape[0], n_feat_1.shape[0], n_feat_2.shape[0]

    e_repr_2 = _edge_mlp(e_feat_2, r_feat_2, n_feat_2, We_2, be_2,
                         block_rows=1000)
    agg_2 = _segment_sum(e_repr_2, dst_2, N1)
    e_repr_1 = _node_edge_mlp(n_feat_1, agg_2, e_feat_1, r_feat_1,
                              W1_2, b1_2, g1_2, bt1_2, W2_2, b2_2, g2_2, bt2_2,
                              We_1, be_1, block_rows=1000)
    agg_1 = _segment_sum(e_repr_1, dst_1, N0)
    n0 = _node_mlp(n_feat_0, agg_1, W1_1, b1_1, g1_1, bt1_1, W2_1, b2_1,
                   g2_1, bt2_1, block_rows=1000)
    return n0


# speedup vs baseline: 1.1289x; 1.0892x over previous
import functools

import jax
import jax.numpy as jnp
from jax import lax
from jax.experimental import pallas as pl
from jax.experimental.pallas import tpu as pltpu
from jax.experimental.pallas import tpu_sc as plsc

H = 256


def _ln(x, g, b, eps=1e-5):
    m = jnp.mean(x, axis=-1, keepdims=True)
    v = jnp.mean((x - m) ** 2, axis=-1, keepdims=True)
    return (x - m) * jax.lax.rsqrt(v + eps) * g + b


def _edge_mlp_body(ef_ref, rf_ref, nf_ref, We_ref, be_ref, out_ref):
    acc = jnp.dot(ef_ref[...], We_ref[0:H, :], preferred_element_type=jnp.float32)
    acc += jnp.dot(rf_ref[...], We_ref[H:2 * H, :], preferred_element_type=jnp.float32)
    acc += jnp.dot(nf_ref[...], We_ref[2 * H:3 * H, :], preferred_element_type=jnp.float32)
    out_ref[...] = jnp.maximum(acc + be_ref[...], 0.0)


def _edge_mlp(ef, rf, nf, We, be, block_rows):
    E = ef.shape[0]
    assert E % block_rows == 0
    grid = (E // block_rows,)
    row_spec = pl.BlockSpec((block_rows, H), lambda i: (i, 0))
    full_w = pl.BlockSpec((3 * H, H), lambda i: (0, 0))
    vec = pl.BlockSpec((H,), lambda i: (0,))
    return pl.pallas_call(
        _edge_mlp_body,
        grid=grid,
        in_specs=[row_spec, row_spec, row_spec, full_w, vec],
        out_specs=row_spec,
        out_shape=jax.ShapeDtypeStruct((E, H), jnp.float32),
    )(ef, rf, nf, We, be)


def _node_mlp_block(nf, agg, W1_ref, b1_ref, g1_ref, bt1_ref, W2_ref, b2_ref,
                    g2_ref, bt2_ref):
    m = jnp.dot(nf, W1_ref[0:H, :], preferred_element_type=jnp.float32)
    m += jnp.dot(agg, W1_ref[H:2 * H, :], preferred_element_type=jnp.float32)
    h = jnp.maximum(_ln(m + b1_ref[...], g1_ref[...], bt1_ref[...]), 0.0)
    h2 = jnp.dot(h, W2_ref[...], preferred_element_type=jnp.float32) + b2_ref[...]
    return jnp.maximum(_ln(h2, g2_ref[...], bt2_ref[...]), 0.0)


def _node_edge_body(nf1_ref, agg_ref, ef_ref, rf_ref,
                    W1_ref, b1_ref, g1_ref, bt1_ref, W2_ref, b2_ref, g2_ref,
                    bt2_ref, We_ref, be_ref, out_ref):
    n1 = _node_mlp_block(nf1_ref[...], agg_ref[...], W1_ref, b1_ref, g1_ref,
                         bt1_ref, W2_ref, b2_ref, g2_ref, bt2_ref)
    acc = jnp.dot(ef_ref[...], We_ref[0:H, :], preferred_element_type=jnp.float32)
    acc += jnp.dot(rf_ref[...], We_ref[H:2 * H, :], preferred_element_type=jnp.float32)
    acc += jnp.dot(n1, We_ref[2 * H:3 * H, :], preferred_element_type=jnp.float32)
    out_ref[...] = jnp.maximum(acc + be_ref[...], 0.0)


def _node_edge_mlp(nf1, agg, ef, rf, W1, b1, g1, bt1, W2, b2, g2, bt2, We, be,
                   block_rows):
    S = nf1.shape[0]
    assert S % block_rows == 0
    grid = (S // block_rows,)
    row_spec = pl.BlockSpec((block_rows, H), lambda i: (i, 0))
    w2h = pl.BlockSpec((2 * H, H), lambda i: (0, 0))
    w1h = pl.BlockSpec((H, H), lambda i: (0, 0))
    w3h = pl.BlockSpec((3 * H, H), lambda i: (0, 0))
    vec = pl.BlockSpec((H,), lambda i: (0,))
    return pl.pallas_call(
        _node_edge_body,
        grid=grid,
        in_specs=[row_spec, row_spec, row_spec, row_spec,
                  w2h, vec, vec, vec, w1h, vec, vec, vec, w3h, vec],
        out_specs=row_spec,
        out_shape=jax.ShapeDtypeStruct((S, H), jnp.float32),
    )(nf1, agg, ef, rf, W1, b1, g1, bt1, W2, b2, g2, bt2, We, be)


def _node_body(nf_ref, agg_ref, W1_ref, b1_ref, g1_ref, bt1_ref, W2_ref,
               b2_ref, g2_ref, bt2_ref, out_ref):
    out_ref[...] = _node_mlp_block(nf_ref[...], agg_ref[...], W1_ref, b1_ref,
                                   g1_ref, bt1_ref, W2_ref, b2_ref, g2_ref,
                                   bt2_ref)


def _node_mlp(nf, agg, W1, b1, g1, bt1, W2, b2, g2, bt2, block_rows):
    S = nf.shape[0]
    assert S % block_rows == 0
    grid = (S // block_rows,)
    row_spec = pl.BlockSpec((block_rows, H), lambda i: (i, 0))
    w2h = pl.BlockSpec((2 * H, H), lambda i: (0, 0))
    w1h = pl.BlockSpec((H, H), lambda i: (0, 0))
    vec = pl.BlockSpec((H,), lambda i: (0,))
    return pl.pallas_call(
        _node_body,
        grid=grid,
        in_specs=[row_spec, row_spec, w2h, vec, vec, vec, w1h, vec, vec, vec],
        out_specs=row_spec,
        out_shape=jax.ShapeDtypeStruct((S, H), jnp.float32),
    )(nf, agg, W1, b1, g1, bt1, W2, b2, g2, bt2)


def _make_seg_sum_sc(E, E_pad, S, G=64, SUB=224):
    NT = 32
    SLAB = E_pad // 16
    n_vregs = SLAB // 16
    S_pad = -(-S // 256) * 256
    OWN = S_pad // NT
    while OWN % SUB:
        SUB -= 8
    n_sub = OWN // SUB
    SH = 18
    MASK18 = (1 << SH) - 1
    DUMPLOC = (1 << 13) - 1
    DUMPPACK = DUMPLOC << SH
    C = 256
    LBUF = (SLAB // C) * C
    LREG = 16 * (SLAB + C) + LBUF
    BUFCAP = max(SLAB + C, LBUF + C)

    assert E_pad % 256 == 0 and E_pad <= (1 << SH)
    assert OWN % 8 == 0 and SUB % 8 == 0 and OWN + 1 < DUMPLOC

    mesh = plsc.VectorSubcoreMesh(core_axis_name="c", subcore_axis_name="s")

    @functools.partial(
        pl.kernel,
        out_type=(jax.ShapeDtypeStruct((S_pad * H,), jnp.float32),
                  jax.ShapeDtypeStruct((NT * LREG,), jnp.int32)),
        mesh=mesh,
        scratch_types=[
            pltpu.VMEM((SLAB,), jnp.int32),
            pltpu.VMEM((LBUF,), jnp.int32),
            pltpu.VMEM((SLAB + C,), jnp.int32),
            pltpu.VMEM((G,), jnp.int32),
            pltpu.VMEM((G,), jnp.int32),
            pltpu.VMEM((G, H), jnp.float32),
            pltpu.VMEM((G, H), jnp.float32),
            pltpu.VMEM(((SUB + 8) * H,), jnp.float32),
            pltpu.SemaphoreType.DMA,
            pltpu.SemaphoreType.DMA,
        ],
        compiler_params=pltpu.CompilerParams(needs_layout_passes=False),
    )
    def seg(x_hbm, dst_hbm, out_hbm, list_hbm, dstbuf, listbuf, bufB,
            idxstage0, idxstage1, rowbuf0, rowbuf1, acc, sem0, sem1):
        c = lax.axis_index("c")
        s = lax.axis_index("s")
        w = c * 16 + s
        lo = w * OWN
        myreg = w * LREG

        iota = lax.iota(jnp.int32, 16)
        fifteen = jnp.full((16,), 15, jnp.int32)
        zi = jnp.zeros((16,), jnp.int32)
        zf = jnp.zeros((16,), jnp.float32)
        dumpvec = jnp.full((16,), DUMPPACK, jnp.int32)

        def _chunk(ch, gcur):
            pltpu.sync_copy(dst_hbm.at[pl.ds(ch * SLAB, SLAB)], dstbuf)

            def _cbody(v, cur):
                d = dstbuf[pl.ds(v * 16, 16)]
                m = (d >= lo) & (d < lo + OWN)
                mi = jnp.where(m, jnp.int32(1), jnp.int32(0))
                csum = plsc.cumsum(mi)
                pos = cur + csum - 1
                eidx = iota + (ch * SLAB + v * 16)
                packed = eidx | ((d - lo) << SH)
                plsc.store_scatter(bufB, [pos], packed, mask=m)
                return cur + jnp.max(csum)
            cur = lax.fori_loop(0, n_vregs, _cbody, jnp.int32(0))

            for k in range(C // 16):
                bufB[pl.ds(cur + k * 16, 16)] = dumpvec
            n_c = (cur + C - 1) // C

            def _flush(i, _):
                pltpu.sync_copy(
                    bufB.at[pl.ds(i * C, C)],
                    list_hbm.at[pl.ds(
                        pl.multiple_of(myreg + gcur + i * C, C), C)])
                return 0
            lax.fori_loop(0, n_c, _flush, 0)
            return gcur + n_c * C
        gcur = lax.fori_loop(0, 16, _chunk, jnp.int32(0))

        gv = zi + gcur

        sub_size = SUB
        def _sub(p, _):
            sub_base = p * SUB

            def _zacc(i, _):
                for k in range(H // 16):
                    acc[pl.ds(i * H + k * 16, 16)] = zf
                return 0
            lax.fori_loop(0, sub_size, _zacc, 0)

            n_l = (gcur + LBUF - 1) // LBUF

            def _lchunk(i, _):
                pltpu.sync_copy(
                    list_hbm.at[pl.ds(
                        pl.multiple_of(myreg + i * LBUF, C), LBUF)],
                    listbuf)

                def _sbody(v, cur):
                    pk = listbuf[pl.ds(v * 16, 16)]
                    lvt = pk >> SH
                    ei = iota + (i * LBUF + v * 16)
                    m = ((lvt >= sub_base) & (lvt < sub_base + sub_size)
                         & (ei < gv))
                    mi = jnp.where(m, jnp.int32(1), jnp.int32(0))
                    csum = plsc.cumsum(mi)
                    pos = cur + csum - 1
                    plsc.store_scatter(bufB, [pos], pk, mask=m)
                    return cur + jnp.max(csum)
                cur = lax.fori_loop(0, LBUF // 16, _sbody, jnp.int32(0))

                for k in range(G // 16):
                    bufB[pl.ds(cur + k * 16, 16)] = dumpvec
                n_g = (cur + G - 1) // G

                def _prep_start(g, idxs, rb, sm):
                    for k in range(G // 16):
                        pk = bufB[pl.ds(g * G + k * 16, 16)]
                        idxs[pl.ds(k * 16, 16)] = pk & MASK18
                    pltpu.async_copy(x_hbm.at[idxs], rb, sm)

                def _accum(g, idxs, rb, sm):
                    pltpu.make_async_copy(x_hbm.at[idxs], rb, sm).wait()

                    def _grp(k, _):
                        pk16 = bufB[pl.ds(g * G + k * 16, 16)]
                        lv16 = (pk16 >> SH) - sub_base
                        lv16 = jnp.where(
                            (lv16 >= 0) & (lv16 < sub_size), lv16, SUB)
                        for e in range(16):
                            lv = lv16[e]
                            base = pl.multiple_of(lv * H, H)
                            row = k * 16 + e
                            for cc in range(H // 16):
                                sl = pl.ds(base + cc * 16, 16)
                                acc[sl] += rb[row, pl.ds(cc * 16, 16)]
                        return 0
                    lax.fori_loop(0, G // 16, _grp, 0)

                @pl.when(n_g > 0)
                def _():
                    _prep_start(0, idxstage0, rowbuf0, sem0)

                def _gbody(g, _):
                    even = (g % 2) == 0

                    @pl.when((g + 1 < n_g) & even)
                    def _():
                        _prep_start(g + 1, idxstage1, rowbuf1, sem1)

                    @pl.when((g + 1 < n_g) & jnp.logical_not(even))
                    def _():
                        _prep_start(g + 1, idxstage0, rowbuf0, sem0)

                    @pl.when(even)
                    def _():
                        _accum(g, idxstage0, rowbuf0, sem0)

                    @pl.when(jnp.logical_not(even))
                    def _():
                        _accum(g, idxstage1, rowbuf1, sem1)
                    return 0
                lax.fori_loop(0, n_g, _gbody, 0)
                return 0
            lax.fori_loop(0, n_l, _lchunk, 0)

            pltpu.sync_copy(
                acc.at[pl.ds(0, sub_size * H)],
                out_hbm.at[pl.ds(
                    pl.multiple_of((lo + sub_base) * H, H), sub_size * H)])
            return 0
        lax.fori_loop(0, n_sub, _sub, 0)

    return seg, S_pad


def _segment_sum(x, dst, num_segments):
    E = x.shape[0]
    E_pad = -(-E // 256) * 256
    seg, S_pad = _make_seg_sum_sc(E, E_pad, num_segments)
    dst_pad = jnp.concatenate(
        [dst, jnp.full((E_pad - E,), S_pad, jnp.int32)]) if E_pad > E else dst
    out, _ = seg(x, dst_pad)
    return out.reshape(S_pad, H)[:num_segments]


def kernel(n_feat_0, n_feat_1, n_feat_2, e_feat_1, e_feat_2, r_feat_1,
           r_feat_2, dst_1, dst_2, We_1, be_1, W1_1, b1_1, g1_1, bt1_1, W2_1,
           b2_1, g2_1, bt2_1, We_2, be_2, W1_2, b1_2, g1_2, bt1_2, W2_2, b2_2,
           g2_2, bt2_2):
    N0, N1, N2 = n_feat_0.shape[0], n_feat_1.shape[0], n_feat_2.shape[0]

    e_repr_2 = _edge_mlp(e_feat_2, r_feat_2, n_feat_2, We_2, be_2,
                         block_rows=1000)
    agg_2 = _segment_sum(e_repr_2, dst_2, N1)
    e_repr_1 = _node_edge_mlp(n_feat_1, agg_2, e_feat_1, r_feat_1,
                              W1_2, b1_2, g1_2, bt1_2, W2_2, b2_2, g2_2, bt2_2,
                              We_1, be_1, block_rows=1000)
    agg_1 = _segment_sum(e_repr_1, dst_1, N0)
    n0 = _node_mlp(n_feat_0, agg_1, W1_1, b1_1, g1_1, bt1_1, W2_1, b2_1,
                   g2_1, bt2_1, block_rows=1000)
    return n0


# speedup vs baseline: 1.1602x; 1.0277x over previous
import functools

import jax
import jax.numpy as jnp
from jax import lax
from jax.experimental import pallas as pl
from jax.experimental.pallas import tpu as pltpu
from jax.experimental.pallas import tpu_sc as plsc

H = 256


def _ln(x, g, b, eps=1e-5):
    m = jnp.mean(x, axis=-1, keepdims=True)
    v = jnp.mean((x - m) ** 2, axis=-1, keepdims=True)
    return (x - m) * jax.lax.rsqrt(v + eps) * g + b


def _edge_mlp_body(ef_ref, rf_ref, nf_ref, We_ref, be_ref, out_ref):
    acc = jnp.dot(ef_ref[...], We_ref[0:H, :], preferred_element_type=jnp.float32)
    acc += jnp.dot(rf_ref[...], We_ref[H:2 * H, :], preferred_element_type=jnp.float32)
    acc += jnp.dot(nf_ref[...], We_ref[2 * H:3 * H, :], preferred_element_type=jnp.float32)
    out_ref[...] = jnp.maximum(acc + be_ref[...], 0.0)


def _edge_mlp(ef, rf, nf, We, be, block_rows):
    E = ef.shape[0]
    assert E % block_rows == 0
    grid = (E // block_rows,)
    row_spec = pl.BlockSpec((block_rows, H), lambda i: (i, 0))
    full_w = pl.BlockSpec((3 * H, H), lambda i: (0, 0))
    vec = pl.BlockSpec((H,), lambda i: (0,))
    return pl.pallas_call(
        _edge_mlp_body,
        grid=grid,
        in_specs=[row_spec, row_spec, row_spec, full_w, vec],
        out_specs=row_spec,
        out_shape=jax.ShapeDtypeStruct((E, H), jnp.float32),
    )(ef, rf, nf, We, be)


def _node_mlp_block(nf, agg, W1_ref, b1_ref, g1_ref, bt1_ref, W2_ref, b2_ref,
                    g2_ref, bt2_ref):
    m = jnp.dot(nf, W1_ref[0:H, :], preferred_element_type=jnp.float32)
    m += jnp.dot(agg, W1_ref[H:2 * H, :], preferred_element_type=jnp.float32)
    h = jnp.maximum(_ln(m + b1_ref[...], g1_ref[...], bt1_ref[...]), 0.0)
    h2 = jnp.dot(h, W2_ref[...], preferred_element_type=jnp.float32) + b2_ref[...]
    return jnp.maximum(_ln(h2, g2_ref[...], bt2_ref[...]), 0.0)


def _node_edge_body(nf1_ref, agg_ref, ef_ref, rf_ref,
                    W1_ref, b1_ref, g1_ref, bt1_ref, W2_ref, b2_ref, g2_ref,
                    bt2_ref, We_ref, be_ref, out_ref):
    n1 = _node_mlp_block(nf1_ref[...], agg_ref[...], W1_ref, b1_ref, g1_ref,
                         bt1_ref, W2_ref, b2_ref, g2_ref, bt2_ref)
    acc = jnp.dot(ef_ref[...], We_ref[0:H, :], preferred_element_type=jnp.float32)
    acc += jnp.dot(rf_ref[...], We_ref[H:2 * H, :], preferred_element_type=jnp.float32)
    acc += jnp.dot(n1, We_ref[2 * H:3 * H, :], preferred_element_type=jnp.float32)
    out_ref[...] = jnp.maximum(acc + be_ref[...], 0.0)


def _node_edge_mlp(nf1, agg, ef, rf, W1, b1, g1, bt1, W2, b2, g2, bt2, We, be,
                   block_rows):
    S = nf1.shape[0]
    assert S % block_rows == 0
    grid = (S // block_rows,)
    row_spec = pl.BlockSpec((block_rows, H), lambda i: (i, 0))
    w2h = pl.BlockSpec((2 * H, H), lambda i: (0, 0))
    w1h = pl.BlockSpec((H, H), lambda i: (0, 0))
    w3h = pl.BlockSpec((3 * H, H), lambda i: (0, 0))
    vec = pl.BlockSpec((H,), lambda i: (0,))
    return pl.pallas_call(
        _node_edge_body,
        grid=grid,
        in_specs=[row_spec, row_spec, row_spec, row_spec,
                  w2h, vec, vec, vec, w1h, vec, vec, vec, w3h, vec],
        out_specs=row_spec,
        out_shape=jax.ShapeDtypeStruct((S, H), jnp.float32),
    )(nf1, agg, ef, rf, W1, b1, g1, bt1, W2, b2, g2, bt2, We, be)


def _node_body(nf_ref, agg_ref, W1_ref, b1_ref, g1_ref, bt1_ref, W2_ref,
               b2_ref, g2_ref, bt2_ref, out_ref):
    out_ref[...] = _node_mlp_block(nf_ref[...], agg_ref[...], W1_ref, b1_ref,
                                   g1_ref, bt1_ref, W2_ref, b2_ref, g2_ref,
                                   bt2_ref)


def _node_mlp(nf, agg, W1, b1, g1, bt1, W2, b2, g2, bt2, block_rows):
    S = nf.shape[0]
    assert S % block_rows == 0
    grid = (S // block_rows,)
    row_spec = pl.BlockSpec((block_rows, H), lambda i: (i, 0))
    w2h = pl.BlockSpec((2 * H, H), lambda i: (0, 0))
    w1h = pl.BlockSpec((H, H), lambda i: (0, 0))
    vec = pl.BlockSpec((H,), lambda i: (0,))
    return pl.pallas_call(
        _node_body,
        grid=grid,
        in_specs=[row_spec, row_spec, w2h, vec, vec, vec, w1h, vec, vec, vec],
        out_specs=row_spec,
        out_shape=jax.ShapeDtypeStruct((S, H), jnp.float32),
    )(nf, agg, W1, b1, g1, bt1, W2, b2, g2, bt2)


def _make_seg_sum_sc(E, E_pad, S, G=64, SUB=224):
    NT = 32
    SLAB = E_pad // 16
    n_vregs = SLAB // 16
    S_pad = -(-S // 256) * 256
    OWN = S_pad // NT
    while OWN % SUB:
        SUB -= 8
    n_sub = OWN // SUB
    SH = 18
    MASK18 = (1 << SH) - 1
    DUMPLOC = (1 << 13) - 1
    DUMPPACK = DUMPLOC << SH
    C = 256
    LBUF = (SLAB // C) * C
    LREG = 16 * (SLAB + C) + LBUF
    BUFCAP = max(SLAB + C, LBUF + C)

    assert E_pad % 256 == 0 and E_pad <= (1 << SH)
    assert OWN % 8 == 0 and SUB % 8 == 0 and OWN + 1 < DUMPLOC

    mesh = plsc.VectorSubcoreMesh(core_axis_name="c", subcore_axis_name="s")

    @functools.partial(
        pl.kernel,
        out_type=(jax.ShapeDtypeStruct((S_pad * H,), jnp.float32),
                  jax.ShapeDtypeStruct((NT * LREG,), jnp.int32)),
        mesh=mesh,
        scratch_types=[
            pltpu.VMEM((SLAB,), jnp.int32),
            pltpu.VMEM((LBUF,), jnp.int32),
            pltpu.VMEM((SLAB + C,), jnp.int32),
            pltpu.VMEM((G,), jnp.int32),
            pltpu.VMEM((G,), jnp.int32),
            pltpu.VMEM((G, H), jnp.float32),
            pltpu.VMEM((G, H), jnp.float32),
            pltpu.VMEM(((SUB + 8) * H,), jnp.float32),
            pltpu.SemaphoreType.DMA,
            pltpu.SemaphoreType.DMA,
        ],
        compiler_params=pltpu.CompilerParams(needs_layout_passes=False),
    )
    def seg(x_hbm, dst_hbm, out_hbm, list_hbm, dstbuf, listbuf, bufB,
            idxstage0, idxstage1, rowbuf0, rowbuf1, acc, sem0, sem1):
        c = lax.axis_index("c")
        s = lax.axis_index("s")
        w = c * 16 + s
        lo = w * OWN
        myreg = w * LREG

        iota = lax.iota(jnp.int32, 16)
        fifteen = jnp.full((16,), 15, jnp.int32)
        zi = jnp.zeros((16,), jnp.int32)
        zf = jnp.zeros((16,), jnp.float32)
        dumpvec = jnp.full((16,), DUMPPACK, jnp.int32)

        def _chunk(ch, gcur):
            pltpu.sync_copy(dst_hbm.at[pl.ds(ch * SLAB, SLAB)], dstbuf)

            def _cbody(v, cur):
                d = dstbuf[pl.ds(v * 16, 16)]
                m = (d >= lo) & (d < lo + OWN)
                mi = jnp.where(m, jnp.int32(1), jnp.int32(0))
                csum = plsc.cumsum(mi)
                pos = cur + csum - 1
                eidx = iota + (ch * SLAB + v * 16)
                packed = eidx | ((d - lo) << SH)
                plsc.store_scatter(bufB, [pos], packed, mask=m)
                return cur + csum[15]
            cur = lax.fori_loop(0, n_vregs, _cbody, jnp.int32(0))

            for k in range(C // 16):
                bufB[pl.ds(cur + k * 16, 16)] = dumpvec
            n_c = (cur + C - 1) // C

            def _flush(i, _):
                pltpu.sync_copy(
                    bufB.at[pl.ds(i * C, C)],
                    list_hbm.at[pl.ds(
                        pl.multiple_of(myreg + gcur + i * C, C), C)])
                return 0
            lax.fori_loop(0, n_c, _flush, 0)
            return gcur + n_c * C
        gcur = lax.fori_loop(0, 16, _chunk, jnp.int32(0))

        gv = zi + gcur

        sub_size = SUB
        def _sub(p, _):
            sub_base = p * SUB

            def _zacc(i, _):
                for k in range(H // 16):
                    acc[pl.ds(i * H + k * 16, 16)] = zf
                return 0
            lax.fori_loop(0, sub_size, _zacc, 0)

            n_l = (gcur + LBUF - 1) // LBUF

            def _lchunk(i, _):
                pltpu.sync_copy(
                    list_hbm.at[pl.ds(
                        pl.multiple_of(myreg + i * LBUF, C), LBUF)],
                    listbuf)

                def _sbody(v, cur):
                    pk = listbuf[pl.ds(v * 16, 16)]
                    lvt = pk >> SH
                    ei = iota + (i * LBUF + v * 16)
                    m = ((lvt >= sub_base) & (lvt < sub_base + sub_size)
                         & (ei < gv))
                    mi = jnp.where(m, jnp.int32(1), jnp.int32(0))
                    csum = plsc.cumsum(mi)
                    pos = cur + csum - 1
                    plsc.store_scatter(bufB, [pos], pk, mask=m)
                    return cur + csum[15]
                cur = lax.fori_loop(0, LBUF // 16, _sbody, jnp.int32(0))

                for k in range(G // 16):
                    bufB[pl.ds(cur + k * 16, 16)] = dumpvec
                n_g = (cur + G - 1) // G

                def _prep_start(g, idxs, rb, sm):
                    for k in range(G // 16):
                        pk = bufB[pl.ds(g * G + k * 16, 16)]
                        idxs[pl.ds(k * 16, 16)] = pk & MASK18
                    pltpu.async_copy(x_hbm.at[idxs], rb, sm)

                def _accum(g, idxs, rb, sm):
                    pltpu.make_async_copy(x_hbm.at[idxs], rb, sm).wait()

                    def _grp(k, _):
                        pk16 = bufB[pl.ds(g * G + k * 16, 16)]
                        lv16 = (pk16 >> SH) - sub_base
                        lv16 = jnp.where(
                            (lv16 >= 0) & (lv16 < sub_size), lv16, SUB)
                        for e in range(16):
                            lv = lv16[e]
                            base = pl.multiple_of(lv * H, H)
                            row = k * 16 + e
                            for cc in range(H // 16):
                                sl = pl.ds(base + cc * 16, 16)
                                acc[sl] += rb[row, pl.ds(cc * 16, 16)]
                        return 0
                    lax.fori_loop(0, G // 16, _grp, 0)

                @pl.when(n_g > 0)
                def _():
                    _prep_start(0, idxstage0, rowbuf0, sem0)

                def _gbody(g, _):
                    even = (g % 2) == 0

                    @pl.when((g + 1 < n_g) & even)
                    def _():
                        _prep_start(g + 1, idxstage1, rowbuf1, sem1)

                    @pl.when((g + 1 < n_g) & jnp.logical_not(even))
                    def _():
                        _prep_start(g + 1, idxstage0, rowbuf0, sem0)

                    @pl.when(even)
                    def _():
                        _accum(g, idxstage0, rowbuf0, sem0)

                    @pl.when(jnp.logical_not(even))
                    def _():
                        _accum(g, idxstage1, rowbuf1, sem1)
                    return 0
                lax.fori_loop(0, n_g, _gbody, 0)
                return 0
            lax.fori_loop(0, n_l, _lchunk, 0)

            pltpu.sync_copy(
                acc.at[pl.ds(0, sub_size * H)],
                out_hbm.at[pl.ds(
                    pl.multiple_of((lo + sub_base) * H, H), sub_size * H)])
            return 0
        lax.fori_loop(0, n_sub, _sub, 0)

    return seg, S_pad


def _segment_sum(x, dst, num_segments):
    E = x.shape[0]
    E_pad = -(-E // 256) * 256
    seg, S_pad = _make_seg_sum_sc(E, E_pad, num_segments)
    dst_pad = jnp.concatenate(
        [dst, jnp.full((E_pad - E,), S_pad, jnp.int32)]) if E_pad > E else dst
    out, _ = seg(x, dst_pad)
    return out.reshape(S_pad, H)[:num_segments]


def kernel(n_feat_0, n_feat_1, n_feat_2, e_feat_1, e_feat_2, r_feat_1,
           r_feat_2, dst_1, dst_2, We_1, be_1, W1_1, b1_1, g1_1, bt1_1, W2_1,
           b2_1, g2_1, bt2_1, We_2, be_2, W1_2, b1_2, g1_2, bt1_2, W2_2, b2_2,
           g2_2, bt2_2):
    N0, N1, N2 = n_feat_0.shape[0], n_feat_1.shape[0], n_feat_2.shape[0]

    e_repr_2 = _edge_mlp(e_feat_2, r_feat_2, n_feat_2, We_2, be_2,
                         block_rows=1000)
    agg_2 = _segment_sum(e_repr_2, dst_2, N1)
    e_repr_1 = _node_edge_mlp(n_feat_1, agg_2, e_feat_1, r_feat_1,
                              W1_2, b1_2, g1_2, bt1_2, W2_2, b2_2, g2_2, bt2_2,
                              We_1, be_1, block_rows=1000)
    agg_1 = _segment_sum(e_repr_1, dst_1, N0)
    n0 = _node_mlp(n_feat_0, agg_1, W1_1, b1_1, g1_1, bt1_1, W2_1, b2_1,
                   g2_1, bt2_1, block_rows=1000)
    return n0
